# baseline (speedup 1.0000x reference)
import functools

import jax
import jax.numpy as jnp
from jax import lax
from jax.experimental import pallas as pl
from jax.experimental.pallas import tpu as pltpu
from jax.experimental.pallas import tpu_sc as plsc

NC = 2
NS = 16
NW = NC * NS
SUB = 128


def _sc_gather(idx2d, QL, exer_emb_w, batch):
    qlw = QL.shape[1]
    emb_dim = exer_emb_w.shape[1]
    per_w = batch // NW
    n_sub = per_w // SUB

    mesh = plsc.VectorSubcoreMesh(core_axis_name="c", subcore_axis_name="s")

    @functools.partial(
        pl.kernel,
        out_type=(
            jax.ShapeDtypeStruct((batch, qlw), jnp.float32),
            jax.ShapeDtypeStruct((batch, emb_dim), jnp.float32),
        ),
        mesh=mesh,
        compiler_params=pltpu.CompilerParams(use_tc_tiling_on_sc=False),
        scratch_types=[
            pltpu.VMEM((n_sub, SUB), jnp.int32),
            pltpu.VMEM((per_w, qlw), jnp.float32),
            pltpu.VMEM((per_w, emb_dim), jnp.float32),
            pltpu.SemaphoreType.DMA,
            pltpu.SemaphoreType.DMA,
        ],
    )
    def gather_kernel(idx_hbm, ql_hbm, e_hbm, qlg_hbm, eg_hbm,
                      idx_v, ql_v, e_v, sem_q, sem_e):
        wid = lax.axis_index("s") * NC + lax.axis_index("c")
        base = wid * per_w
        pltpu.sync_copy(idx_hbm.at[pl.ds(wid * n_sub, n_sub)], idx_v)
        copies = []
        for j in range(n_sub):
            sl = pl.ds(j * SUB, SUB)
            copies.append(pltpu.async_copy(ql_hbm.at[idx_v.at[j]], ql_v.at[sl], sem_q))
            copies.append(pltpu.async_copy(e_hbm.at[idx_v.at[j]], e_v.at[sl], sem_e))
        for cp in copies:
            cp.wait()
        pltpu.sync_copy(ql_v, qlg_hbm.at[pl.ds(base, per_w)])
        pltpu.sync_copy(e_v, eg_hbm.at[pl.ds(base, per_w)])

    return gather_kernel(idx2d, QL, exer_emb_w)


def _tc_combine(QLg, Eg, concept_emb_w, n_concept):
    batch = QLg.shape[0]
    qlw = QLg.shape[1]
    emb_dim = concept_emb_w.shape[1]
    blk = 2048

    def body(ql_ref, e_ref, w_ref, o_ref):
        ql = ql_ref[...]
        q = ql[:, :n_concept]
        lam = ql[:, n_concept:n_concept + 1]
        ce = lax.dot_general(
            q, w_ref[...], (((1,), (0,)), ((), ())),
            precision=lax.Precision.HIGHEST,
            preferred_element_type=jnp.float32,
        )
        cnt = jnp.sum(q, axis=1, keepdims=True)
        o_ref[...] = e_ref[...] + lam * ce / cnt

    return pl.pallas_call(
        body,
        grid=(batch // blk,),
        in_specs=[
            pl.BlockSpec((blk, qlw), lambda i: (i, 0)),
            pl.BlockSpec((blk, emb_dim), lambda i: (i, 0)),
            pl.BlockSpec((n_concept, emb_dim), lambda i: (0, 0)),
        ],
        out_specs=pl.BlockSpec((blk, emb_dim), lambda i: (i, 0)),
        out_shape=jax.ShapeDtypeStruct((batch, emb_dim), jnp.float32),
    )(QLg, Eg, concept_emb_w)


def kernel(exercise, exer_emb_w, exer_lam_w, concept_emb_w, Q_matrix):
    batch = exercise.shape[0]
    n_ex, n_concept = Q_matrix.shape
    qlw = -(-(n_concept + 1) // 8) * 8
    QL = jnp.concatenate(
        [Q_matrix, exer_lam_w,
         jnp.zeros((n_ex, qlw - n_concept - 1), jnp.float32)], axis=1)
    idx2d = exercise.astype(jnp.int32).reshape(batch // SUB, SUB)
    QLg, Eg = _sc_gather(idx2d, QL, exer_emb_w, batch)
    return _tc_combine(QLg, Eg, concept_emb_w, n_concept)


# speedup vs baseline: 1.2753x; 1.2753x over previous
import functools

import jax
import jax.numpy as jnp
from jax import lax
from jax.experimental import pallas as pl
from jax.experimental.pallas import tpu as pltpu
from jax.experimental.pallas import tpu_sc as plsc

NC = 2
NS = 16
NW = NC * NS
SUB = 128


def _sc_gather(idx2d, QL, exer_emb_w, batch):
    qlw = QL.shape[1]
    emb_dim = exer_emb_w.shape[1]
    per_w = batch // NW
    n_sub = per_w // SUB

    mesh = plsc.VectorSubcoreMesh(core_axis_name="c", subcore_axis_name="s")

    @functools.partial(
        pl.kernel,
        out_type=(
            jax.ShapeDtypeStruct((batch, qlw), jnp.float32),
            jax.ShapeDtypeStruct((batch, emb_dim), jnp.float32),
        ),
        mesh=mesh,
        compiler_params=pltpu.CompilerParams(use_tc_tiling_on_sc=False),
        scratch_types=[
            pltpu.VMEM((n_sub, SUB), jnp.int32),
            pltpu.VMEM((per_w, qlw), jnp.float32),
            pltpu.VMEM((per_w, emb_dim), jnp.float32),
            pltpu.SemaphoreType.DMA,
            pltpu.SemaphoreType.DMA,
        ],
    )
    def gather_kernel(idx_hbm, ql_hbm, e_hbm, qlg_hbm, eg_hbm,
                      idx_v, ql_v, e_v, sem_q, sem_e):
        wid = lax.axis_index("s") * NC + lax.axis_index("c")
        base = wid * per_w
        pltpu.sync_copy(idx_hbm.at[pl.ds(wid * n_sub, n_sub)], idx_v)
        copies = []
        for j in range(n_sub):
            sl = pl.ds(j * SUB, SUB)
            copies.append(pltpu.async_copy(ql_hbm.at[idx_v.at[j]], ql_v.at[sl], sem_q))
            copies.append(pltpu.async_copy(e_hbm.at[idx_v.at[j]], e_v.at[sl], sem_e))
        for cp in copies:
            cp.wait()
        pltpu.sync_copy(ql_v, qlg_hbm.at[pl.ds(base, per_w)])
        pltpu.sync_copy(e_v, eg_hbm.at[pl.ds(base, per_w)])

    return gather_kernel(idx2d, QL, exer_emb_w)


def _tc_combine(QLg, Eg, concept_emb_w, n_concept):
    batch = QLg.shape[0]
    qlw = QLg.shape[1]
    emb_dim = concept_emb_w.shape[1]
    blk = 2048

    def body(ql_ref, e_ref, w_ref, o_ref):
        ql = ql_ref[...]
        q = ql[:, :n_concept]
        lam = ql[:, n_concept:n_concept + 1]
        ce = lax.dot_general(
            q, w_ref[...], (((1,), (0,)), ((), ())),
            precision=lax.Precision.HIGHEST,
            preferred_element_type=jnp.float32,
        )
        cnt = jnp.sum(q, axis=1, keepdims=True)
        o_ref[...] = e_ref[...] + lam * ce / cnt

    return pl.pallas_call(
        body,
        grid=(batch // blk,),
        in_specs=[
            pl.BlockSpec((blk, qlw), lambda i: (i, 0)),
            pl.BlockSpec((blk, emb_dim), lambda i: (i, 0)),
            pl.BlockSpec((n_concept, emb_dim), lambda i: (0, 0)),
        ],
        out_specs=pl.BlockSpec((blk, emb_dim), lambda i: (i, 0)),
        out_shape=jax.ShapeDtypeStruct((batch, emb_dim), jnp.float32),
    )(QLg, Eg, concept_emb_w)


def _tc_pack(Q_matrix, exer_lam_w, qlw):
    n_ex, n_concept = Q_matrix.shape
    rblk = 4000

    def body(q_ref, l_ref, o_ref):
        o_ref[:, :n_concept] = q_ref[...]
        o_ref[:, n_concept:n_concept + 1] = l_ref[...]
        o_ref[:, n_concept + 1:] = jnp.zeros(
            (rblk, qlw - n_concept - 1), jnp.float32)

    return pl.pallas_call(
        body,
        grid=(n_ex // rblk,),
        in_specs=[
            pl.BlockSpec((rblk, n_concept), lambda i: (i, 0)),
            pl.BlockSpec((rblk, 1), lambda i: (i, 0)),
        ],
        out_specs=pl.BlockSpec((rblk, qlw), lambda i: (i, 0)),
        out_shape=jax.ShapeDtypeStruct((n_ex, qlw), jnp.float32),
    )(Q_matrix, exer_lam_w)


def kernel(exercise, exer_emb_w, exer_lam_w, concept_emb_w, Q_matrix):
    batch = exercise.shape[0]
    n_ex, n_concept = Q_matrix.shape
    qlw = -(-(n_concept + 1) // 8) * 8
    QL = _tc_pack(Q_matrix, exer_lam_w, qlw)
    idx2d = exercise.astype(jnp.int32).reshape(batch // SUB, SUB)
    QLg, Eg = _sc_gather(idx2d, QL, exer_emb_w, batch)
    return _tc_combine(QLg, Eg, concept_emb_w, n_concept)


# speedup vs baseline: 2.8338x; 2.2219x over previous
import functools

import jax
import jax.numpy as jnp
from jax import lax
from jax.experimental import pallas as pl
from jax.experimental.pallas import tpu as pltpu
from jax.experimental.pallas import tpu_sc as plsc

NC = 2
NS = 16
NW = NC * NS
SUB = 128
TBLW = 128


def _tc_pack(QT, lamT, embT):
    n_concept, n_ex = QT.shape
    emb_dim = embT.shape[0]
    blk = 2048

    def body(qt_ref, l_ref, e_ref, ql_ref, em_ref):
        ql_ref[:, :n_concept] = jnp.transpose(qt_ref[...])
        ql_ref[:, n_concept:n_concept + 1] = jnp.transpose(l_ref[...])
        ql_ref[:, n_concept + 1:] = jnp.zeros(
            (blk, TBLW - n_concept - 1), jnp.float32)
        em_ref[:, :emb_dim] = jnp.transpose(e_ref[...])
        em_ref[:, emb_dim:] = jnp.zeros((blk, TBLW - emb_dim), jnp.float32)

    return pl.pallas_call(
        body,
        grid=(n_ex // blk,),
        in_specs=[
            pl.BlockSpec((n_concept, blk), lambda i: (0, i)),
            pl.BlockSpec((1, blk), lambda i: (0, i)),
            pl.BlockSpec((emb_dim, blk), lambda i: (0, i)),
        ],
        out_specs=[
            pl.BlockSpec((blk, TBLW), lambda i: (i, 0)),
            pl.BlockSpec((blk, TBLW), lambda i: (i, 0)),
        ],
        out_shape=(
            jax.ShapeDtypeStruct((n_ex, TBLW), jnp.float32),
            jax.ShapeDtypeStruct((n_ex, TBLW), jnp.float32),
        ),
    )(QT, lamT, embT)


def _sc_gather(idx2d, QL, EM, batch):
    per_w = batch // NW
    n_sub = per_w // SUB
    half = per_w // 2

    mesh = plsc.VectorSubcoreMesh(core_axis_name="c", subcore_axis_name="s")

    @functools.partial(
        pl.kernel,
        out_type=(
            jax.ShapeDtypeStruct((batch, TBLW), jnp.float32),
            jax.ShapeDtypeStruct((batch, TBLW), jnp.float32),
        ),
        mesh=mesh,
        compiler_params=pltpu.CompilerParams(use_tc_tiling_on_sc=False),
        scratch_types=[
            pltpu.VMEM((n_sub, SUB), jnp.int32),
            pltpu.VMEM((half, TBLW), jnp.float32),
            pltpu.VMEM((half, TBLW), jnp.float32),
            pltpu.SemaphoreType.DMA,
            pltpu.SemaphoreType.DMA,
        ],
    )
    def gather_kernel(idx_hbm, ql_hbm, em_hbm, qlg_hbm, emg_hbm,
                      idx_v, ql_v, em_v, sem_q, sem_e):
        wid = lax.axis_index("s") * NC + lax.axis_index("c")
        base = wid * per_w
        pltpu.sync_copy(idx_hbm.at[pl.ds(wid * n_sub, n_sub)], idx_v)
        for h in range(2):
            copies = []
            for j in range(n_sub // 2):
                sl = pl.ds(j * SUB, SUB)
                ij = h * (n_sub // 2) + j
                copies.append(pltpu.async_copy(
                    ql_hbm.at[idx_v.at[ij]], ql_v.at[sl], sem_q))
                copies.append(pltpu.async_copy(
                    em_hbm.at[idx_v.at[ij]], em_v.at[sl], sem_e))
            for cp in copies:
                cp.wait()
            pltpu.sync_copy(ql_v, qlg_hbm.at[pl.ds(base + h * half, half)])
            pltpu.sync_copy(em_v, emg_hbm.at[pl.ds(base + h * half, half)])

    return gather_kernel(idx2d, QL, EM)


def _tc_combine(QLg, EMg, concept_emb_w, n_concept):
    batch = QLg.shape[0]
    emb_dim = concept_emb_w.shape[1]
    blk = 2048

    def body(ql_ref, em_ref, w_ref, o_ref):
        ql = ql_ref[...]
        q = ql[:, :n_concept]
        lam = ql[:, n_concept:n_concept + 1]
        emb = em_ref[:, :emb_dim]
        ce = lax.dot_general(
            q, w_ref[...], (((1,), (0,)), ((), ())),
            precision=lax.Precision.HIGHEST,
            preferred_element_type=jnp.float32,
        )
        cnt = jnp.sum(q, axis=1, keepdims=True)
        o_ref[...] = emb + lam * ce / cnt

    return pl.pallas_call(
        body,
        grid=(batch // blk,),
        in_specs=[
            pl.BlockSpec((blk, TBLW), lambda i: (i, 0)),
            pl.BlockSpec((blk, TBLW), lambda i: (i, 0)),
            pl.BlockSpec((n_concept, emb_dim), lambda i: (0, 0)),
        ],
        out_specs=pl.BlockSpec((blk, emb_dim), lambda i: (i, 0)),
        out_shape=jax.ShapeDtypeStruct((batch, emb_dim), jnp.float32),
    )(QLg, EMg, concept_emb_w)


def kernel(exercise, exer_emb_w, exer_lam_w, concept_emb_w, Q_matrix):
    batch = exercise.shape[0]
    n_concept = Q_matrix.shape[1]
    QL, EM = _tc_pack(Q_matrix.T, exer_lam_w.T, exer_emb_w.T)
    idx2d = exercise.astype(jnp.int32).reshape(batch // SUB, SUB)
    QLg, EMg = _sc_gather(idx2d, QL, EM, batch)
    return _tc_combine(QLg, EMg, concept_emb_w, n_concept)


# speedup vs baseline: 3.2150x; 1.1345x over previous
import functools

import jax
import jax.numpy as jnp
from jax import lax
from jax.experimental import pallas as pl
from jax.experimental.pallas import tpu as pltpu
from jax.experimental.pallas import tpu_sc as plsc

NC = 2
NS = 16
NW = NC * NS
SUB = 128
TBLW = 128


def _tc_table(QT, lamT, embT, W):
    n_concept, n_ex = QT.shape
    emb_dim = W.shape[1]
    blk = 2048

    def body(qt_ref, l_ref, e_ref, w_ref, a_ref):
        qt = qt_ref[...]
        ce = lax.dot_general(
            qt, w_ref[...], (((0,), (0,)), ((), ())),
            precision=lax.Precision.HIGHEST,
            preferred_element_type=jnp.float32,
        )
        cnt = jnp.sum(qt, axis=0, keepdims=True)
        scale = l_ref[...] / cnt
        a_ref[:, :emb_dim] = (
            jnp.transpose(e_ref[...]) + jnp.transpose(scale) * ce)
        a_ref[:, emb_dim:] = jnp.zeros((blk, TBLW - emb_dim), jnp.float32)

    return pl.pallas_call(
        body,
        grid=(pl.cdiv(n_ex, blk),),
        in_specs=[
            pl.BlockSpec((n_concept, blk), lambda i: (0, i)),
            pl.BlockSpec((1, blk), lambda i: (0, i)),
            pl.BlockSpec((emb_dim, blk), lambda i: (0, i)),
            pl.BlockSpec((n_concept, emb_dim), lambda i: (0, 0)),
        ],
        out_specs=pl.BlockSpec((blk, TBLW), lambda i: (i, 0)),
        out_shape=jax.ShapeDtypeStruct((n_ex, TBLW), jnp.float32),
    )(QT, lamT, embT, W)


def _sc_gather(idx2d, A, batch, emb_dim):
    per_w = batch // NW
    n_sub = per_w // SUB

    mesh = plsc.VectorSubcoreMesh(core_axis_name="c", subcore_axis_name="s")

    @functools.partial(
        pl.kernel,
        out_type=jax.ShapeDtypeStruct((batch, emb_dim), jnp.float32),
        mesh=mesh,
        compiler_params=pltpu.CompilerParams(use_tc_tiling_on_sc=False),
        scratch_types=[
            pltpu.VMEM((n_sub, SUB), jnp.int32),
            pltpu.VMEM((per_w, TBLW), jnp.float32),
            pltpu.SemaphoreType.DMA,
        ],
    )
    def gather_kernel(idx_hbm, a_hbm, out_hbm, idx_v, row_v, sem):
        wid = lax.axis_index("s") * NC + lax.axis_index("c")
        base = wid * per_w
        pltpu.sync_copy(idx_hbm.at[pl.ds(wid * n_sub, n_sub)], idx_v)
        copies = []
        for j in range(n_sub):
            copies.append(pltpu.async_copy(
                a_hbm.at[idx_v.at[j]], row_v.at[pl.ds(j * SUB, SUB)], sem))
        for cp in copies:
            cp.wait()
        pltpu.sync_copy(row_v.at[:, pl.ds(0, emb_dim)],
                        out_hbm.at[pl.ds(base, per_w)])

    return gather_kernel(idx2d, A)


def kernel(exercise, exer_emb_w, exer_lam_w, concept_emb_w, Q_matrix):
    batch = exercise.shape[0]
    emb_dim = concept_emb_w.shape[1]
    A = _tc_table(Q_matrix.T, exer_lam_w.T, exer_emb_w.T, concept_emb_w)
    idx2d = exercise.astype(jnp.int32).reshape(batch // SUB, SUB)
    return _sc_gather(idx2d, A, batch, emb_dim)


# speedup vs baseline: 3.5551x; 1.1058x over previous
import functools

import jax
import jax.numpy as jnp
from jax import lax
from jax.experimental import pallas as pl
from jax.experimental.pallas import tpu as pltpu
from jax.experimental.pallas import tpu_sc as plsc

NC = 2
NS = 16
NW = NC * NS
SUB = 128
TBLW = 128


def _tc_table(QT, lamT, embT, W):
    n_concept, n_ex = QT.shape
    emb_dim = W.shape[1]
    blk = 2048

    def body(qt_ref, l_ref, e_ref, w_ref, a_ref):
        qt = qt_ref[...]
        cnt = jnp.sum(qt, axis=0, keepdims=True)
        sq = qt * (l_ref[...] / cnt)
        ce = lax.dot_general(
            sq, w_ref[...], (((0,), (0,)), ((), ())),
            precision=lax.Precision.DEFAULT,
            preferred_element_type=jnp.float32,
        )
        row = lax.broadcasted_iota(jnp.int32, (emb_dim, emb_dim), 0)
        col = lax.broadcasted_iota(jnp.int32, (emb_dim, emb_dim), 1)
        ident = (row == col).astype(jnp.float32)
        et = lax.dot_general(
            e_ref[...], ident, (((0,), (0,)), ((), ())),
            precision=lax.Precision.DEFAULT,
            preferred_element_type=jnp.float32,
        )
        a_ref[:, :emb_dim] = et + ce

    return pl.pallas_call(
        body,
        grid=(pl.cdiv(n_ex, blk),),
        in_specs=[
            pl.BlockSpec((n_concept, blk), lambda i: (0, i)),
            pl.BlockSpec((1, blk), lambda i: (0, i)),
            pl.BlockSpec((emb_dim, blk), lambda i: (0, i)),
            pl.BlockSpec((n_concept, emb_dim), lambda i: (0, 0)),
        ],
        out_specs=pl.BlockSpec((blk, TBLW), lambda i: (i, 0)),
        out_shape=jax.ShapeDtypeStruct((n_ex, TBLW), jnp.float32),
    )(QT, lamT, embT, W)


def _sc_gather(idx2d, A, batch, emb_dim):
    per_w = batch // NW
    n_sub = per_w // SUB

    mesh = plsc.VectorSubcoreMesh(core_axis_name="c", subcore_axis_name="s")

    @functools.partial(
        pl.kernel,
        out_type=jax.ShapeDtypeStruct((batch, emb_dim), jnp.float32),
        mesh=mesh,
        compiler_params=pltpu.CompilerParams(use_tc_tiling_on_sc=False),
        scratch_types=[
            pltpu.VMEM((n_sub, SUB), jnp.int32),
            pltpu.VMEM((per_w, TBLW), jnp.float32),
            pltpu.SemaphoreType.DMA,
        ],
    )
    def gather_kernel(idx_hbm, a_hbm, out_hbm, idx_v, row_v, sem):
        wid = lax.axis_index("s") * NC + lax.axis_index("c")
        base = wid * per_w
        pltpu.sync_copy(idx_hbm.at[pl.ds(wid * n_sub, n_sub)], idx_v)
        copies = []
        for j in range(n_sub):
            copies.append(pltpu.async_copy(
                a_hbm.at[idx_v.at[j]], row_v.at[pl.ds(j * SUB, SUB)], sem))
        for cp in copies:
            cp.wait()
        pltpu.sync_copy(row_v.at[:, pl.ds(0, emb_dim)],
                        out_hbm.at[pl.ds(base, per_w)])

    return gather_kernel(idx2d, A)


def kernel(exercise, exer_emb_w, exer_lam_w, concept_emb_w, Q_matrix):
    batch = exercise.shape[0]
    emb_dim = concept_emb_w.shape[1]
    A = _tc_table(Q_matrix.T, exer_lam_w.T, exer_emb_w.T, concept_emb_w)
    idx2d = exercise.astype(jnp.int32).reshape(batch // SUB, SUB)
    return _sc_gather(idx2d, A, batch, emb_dim)


# speedup vs baseline: 4.5532x; 1.2808x over previous
import functools

import jax
import jax.numpy as jnp
from jax import lax
from jax.experimental import pallas as pl
from jax.experimental.pallas import tpu as pltpu
from jax.experimental.pallas import tpu_sc as plsc

NC = 2
NS = 16
NW = NC * NS
SUB = 128
TBLW = 128


def _tc_table(QT, lamT, embT, W):
    n_concept, n_ex = QT.shape
    emb_dim = W.shape[1]
    blk = 4096

    def body(qt_ref, l_ref, e_ref, w_ref, a_ref):
        qt = qt_ref[...]
        cnt = jnp.sum(qt, axis=0, keepdims=True)
        sq = qt * (l_ref[...] / cnt)
        ce = lax.dot_general(
            sq, w_ref[...], (((0,), (0,)), ((), ())),
            precision=lax.Precision.DEFAULT,
            preferred_element_type=jnp.float32,
        )
        row = lax.broadcasted_iota(jnp.int32, (emb_dim, emb_dim), 0)
        col = lax.broadcasted_iota(jnp.int32, (emb_dim, emb_dim), 1)
        ident = (row == col).astype(jnp.float32)
        et = lax.dot_general(
            e_ref[...], ident, (((0,), (0,)), ((), ())),
            precision=lax.Precision.DEFAULT,
            preferred_element_type=jnp.float32,
        )
        a_ref[:, :emb_dim] = et + ce

    return pl.pallas_call(
        body,
        grid=(pl.cdiv(n_ex, blk),),
        in_specs=[
            pl.BlockSpec((n_concept, blk), lambda i: (0, i)),
            pl.BlockSpec((1, blk), lambda i: (0, i)),
            pl.BlockSpec((emb_dim, blk), lambda i: (0, i)),
            pl.BlockSpec((n_concept, emb_dim), lambda i: (0, 0)),
        ],
        out_specs=pl.BlockSpec((blk, TBLW), lambda i: (i, 0)),
        out_shape=jax.ShapeDtypeStruct((n_ex, TBLW), jnp.float32),
    )(QT, lamT, embT, W)


def _sc_gather(idx2d, A, batch, emb_dim):
    per_w = batch // NW
    n_sub = per_w // SUB

    mesh = plsc.VectorSubcoreMesh(core_axis_name="c", subcore_axis_name="s")

    @functools.partial(
        pl.kernel,
        out_type=jax.ShapeDtypeStruct((batch, TBLW), jnp.float32),
        mesh=mesh,
        compiler_params=pltpu.CompilerParams(use_tc_tiling_on_sc=False),
        scratch_types=[
            pltpu.VMEM((n_sub, SUB), jnp.int32),
            pltpu.VMEM((per_w, TBLW), jnp.float32),
            pltpu.SemaphoreType.DMA,
        ],
    )
    def gather_kernel(idx_hbm, a_hbm, out_hbm, idx_v, row_v, sem):
        wid = lax.axis_index("s") * NC + lax.axis_index("c")
        base = wid * per_w
        pltpu.sync_copy(idx_hbm.at[pl.ds(wid * n_sub, n_sub)], idx_v)
        copies = []
        for j in range(n_sub):
            copies.append(pltpu.async_copy(
                a_hbm.at[idx_v.at[j]], row_v.at[pl.ds(j * SUB, SUB)], sem))
        for cp in copies:
            cp.wait()
        pltpu.sync_copy(row_v, out_hbm.at[pl.ds(base, per_w)])

    return gather_kernel(idx2d, A)


def _tc_emit(Ag, emb_dim):
    batch = Ag.shape[0]
    blk = 4096

    def body(a_ref, o_ref):
        o_ref[...] = jnp.transpose(a_ref[:, :emb_dim])

    return pl.pallas_call(
        body,
        grid=(batch // blk,),
        in_specs=[pl.BlockSpec((blk, TBLW), lambda i: (i, 0))],
        out_specs=pl.BlockSpec((emb_dim, blk), lambda i: (0, i)),
        out_shape=jax.ShapeDtypeStruct((emb_dim, batch), jnp.float32),
    )(Ag)


def kernel(exercise, exer_emb_w, exer_lam_w, concept_emb_w, Q_matrix):
    batch = exercise.shape[0]
    emb_dim = concept_emb_w.shape[1]
    A = _tc_table(Q_matrix.T, exer_lam_w.T, exer_emb_w.T, concept_emb_w)
    idx2d = exercise.astype(jnp.int32).reshape(batch // SUB, SUB)
    Ag = _sc_gather(idx2d, A, batch, emb_dim)
    return _tc_emit(Ag, emb_dim).T


# speedup vs baseline: 5.0059x; 1.0994x over previous
import functools

import jax
import jax.numpy as jnp
from jax import lax
from jax.experimental import pallas as pl
from jax.experimental.pallas import tpu as pltpu
from jax.experimental.pallas import tpu_sc as plsc

NC = 2
NS = 16
NW = NC * NS
SUB = 128
TBLW = 128


def _tc_table(QT, lamT, embT, W):
    n_concept, n_ex = QT.shape
    emb_dim = W.shape[1]
    blk = 8192

    def body(qt_ref, l_ref, e_ref, w_ref, a_ref):
        qt = qt_ref[...]
        cnt = jnp.sum(qt, axis=0, keepdims=True)
        sq = qt * (l_ref[...] / cnt)
        ce = lax.dot_general(
            sq, w_ref[...], (((0,), (0,)), ((), ())),
            precision=lax.Precision.DEFAULT,
            preferred_element_type=jnp.float32,
        )
        row = lax.broadcasted_iota(jnp.int32, (emb_dim, emb_dim), 0)
        col = lax.broadcasted_iota(jnp.int32, (emb_dim, emb_dim), 1)
        ident = (row == col).astype(jnp.float32)
        et = lax.dot_general(
            e_ref[...], ident, (((0,), (0,)), ((), ())),
            precision=lax.Precision.DEFAULT,
            preferred_element_type=jnp.float32,
        )
        a_ref[:, :emb_dim] = et + ce

    return pl.pallas_call(
        body,
        grid=(pl.cdiv(n_ex, blk),),
        in_specs=[
            pl.BlockSpec((n_concept, blk), lambda i: (0, i)),
            pl.BlockSpec((1, blk), lambda i: (0, i)),
            pl.BlockSpec((emb_dim, blk), lambda i: (0, i)),
            pl.BlockSpec((n_concept, emb_dim), lambda i: (0, 0)),
        ],
        out_specs=pl.BlockSpec((blk, TBLW), lambda i: (i, 0)),
        out_shape=jax.ShapeDtypeStruct((n_ex, TBLW), jnp.float32),
    )(QT, lamT, embT, W)


def _sc_gather(idx2d, A, batch, emb_dim):
    per_w = batch // NW
    n_sub = per_w // SUB

    mesh = plsc.VectorSubcoreMesh(core_axis_name="c", subcore_axis_name="s")

    @functools.partial(
        pl.kernel,
        out_type=jax.ShapeDtypeStruct((batch, TBLW), jnp.float32),
        mesh=mesh,
        compiler_params=pltpu.CompilerParams(use_tc_tiling_on_sc=False),
        scratch_types=[
            pltpu.VMEM((n_sub, SUB), jnp.int32),
            pltpu.VMEM((per_w, TBLW), jnp.float32),
            pltpu.SemaphoreType.DMA,
        ],
    )
    def gather_kernel(idx_hbm, a_hbm, out_hbm, idx_v, row_v, sem):
        wid = lax.axis_index("s") * NC + lax.axis_index("c")
        base = wid * per_w
        pltpu.sync_copy(idx_hbm.at[pl.ds(wid * n_sub, n_sub)], idx_v)
        copies = []
        for j in range(n_sub):
            copies.append(pltpu.async_copy(
                a_hbm.at[idx_v.at[j]], row_v.at[pl.ds(j * SUB, SUB)], sem))
        for cp in copies:
            cp.wait()
        pltpu.sync_copy(row_v, out_hbm.at[pl.ds(base, per_w)])

    return gather_kernel(idx2d, A)


def _tc_emit(Ag, emb_dim):
    batch = Ag.shape[0]
    blk = 8192

    def body(a_ref, o_ref):
        o_ref[...] = jnp.transpose(a_ref[:, :emb_dim])

    return pl.pallas_call(
        body,
        grid=(batch // blk,),
        in_specs=[pl.BlockSpec((blk, TBLW), lambda i: (i, 0))],
        out_specs=pl.BlockSpec((emb_dim, blk), lambda i: (0, i)),
        out_shape=jax.ShapeDtypeStruct((emb_dim, batch), jnp.float32),
    )(Ag)


def kernel(exercise, exer_emb_w, exer_lam_w, concept_emb_w, Q_matrix):
    batch = exercise.shape[0]
    emb_dim = concept_emb_w.shape[1]
    A = _tc_table(Q_matrix.T, exer_lam_w.T, exer_emb_w.T, concept_emb_w)
    idx2d = exercise.astype(jnp.int32).reshape(batch // SUB, SUB)
    Ag = _sc_gather(idx2d, A, batch, emb_dim)
    return _tc_emit(Ag, emb_dim).T
